# baseline (speedup 1.0000x reference)
import functools

import jax
import jax.numpy as jnp
from jax import lax
from jax.experimental import pallas as pl
from jax.experimental.pallas import tpu as pltpu
from jax.experimental.pallas import tpu_sc as plsc

_NC = 2
_NS = 16
_LANES = 16


def _build_kernel(N, V, D, NG, C):
    NW = _NC * _NS
    rows_per_w = N // NW
    n_chunks = rows_per_w // C
    W = D // 4
    RPB = 128 // W

    mesh = plsc.VectorSubcoreMesh(
        core_axis_name="c", subcore_axis_name="s",
        num_cores=_NC, num_subcores=_NS)

    @functools.partial(
        pl.kernel,
        out_type=jax.ShapeDtypeStruct((N * D,), jnp.float32),
        mesh=mesh,
        compiler_params=pltpu.CompilerParams(needs_layout_passes=False),
        scratch_types=[
            pltpu.VMEM((rows_per_w + _LANES,), jnp.int32),
            pltpu.VMEM((C,), jnp.int32),
            pltpu.VMEM((NG * C,), jnp.int32),
            pltpu.VMEM((C, 128), jnp.int32),
            pltpu.VMEM((NG * C,), jnp.float32),
            pltpu.VMEM((C * D,), jnp.float32),
            pltpu.SemaphoreType.DMA,
            pltpu.SemaphoreType.DMA,
            pltpu.SemaphoreType.DMA,
        ],
    )
    def dequant(idx_hbm, wblk_hbm, sclf_hbm, out_hbm,
                idx_v, bidx_v, eidx_v, rows_v, scl_v, out_v,
                gsem, ssem, osem):
        wid = lax.axis_index("s") * _NC + lax.axis_index("c")
        base = wid * rows_per_w
        pltpu.sync_copy(idx_hbm.at[pl.ds(base, rows_per_w)],
                        idx_v.at[pl.ds(0, rows_per_w)])

        lane = lax.iota(jnp.int32, _LANES)
        col4 = lane * 4
        glo = lane >> 3
        r_in_reg = lane >> 2
        sub4 = lane & 3

        def do_chunk(g, _):
            cbase = g * C

            def build_bidx(t, _):
                iv = idx_v[pl.ds(cbase + t * _LANES, _LANES)]
                bidx_v[pl.ds(t * _LANES, _LANES)] = iv >> 2
                return 0

            lax.fori_loop(0, C // _LANES, build_bidx, 0, unroll=False)

            def build_eidx(t, _):
                iv = idx_v[pl.ds(cbase + t * 4, _LANES)]
                gi = jnp.take(iv, r_in_reg, axis=0)
                eidx_v[pl.ds(t * _LANES, _LANES)] = gi * NG + sub4
                return 0

            lax.fori_loop(0, NG * C // _LANES, build_eidx, 0, unroll=False)

            cpy_w = pltpu.async_copy(wblk_hbm.at[bidx_v], rows_v, gsem)
            cpy_s = pltpu.async_copy(sclf_hbm.at[eidx_v], scl_v, ssem)
            cpy_w.wait()
            cpy_s.wait()

            def do_row(r, _):
                ivv = idx_v[pl.ds(cbase + r, _LANES)]
                sub32 = (ivv[0] & (RPB - 1)) * W
                rb = jnp.full((_LANES,), r * D, jnp.int32) + col4
                sb = jnp.full((_LANES,), r * NG, jnp.int32) + glo
                s_lo = plsc.load_gather(scl_v, [sb])
                s_hi = plsc.load_gather(scl_v, [sb + 2])
                w0 = rows_v[r, pl.ds(sub32, _LANES)]
                w1 = rows_v[r, pl.ds(sub32 + _LANES, _LANES)]
                for h, (w, s) in enumerate(((w0, s_lo), (w1, s_hi))):
                    for j in range(4):
                        v = (w << (24 - 8 * j)) >> 24
                        y = v.astype(jnp.float32) * s
                        plsc.store_scatter(out_v, [rb + (64 * h + j)], y)
                return 0

            lax.fori_loop(0, C, do_row, 0, unroll=False)
            pltpu.async_copy(
                out_v, out_hbm.at[pl.ds((base + cbase) * D, C * D)],
                osem).wait()
            return 0

        lax.fori_loop(0, n_chunks, do_chunk, 0, unroll=False)

    return dequant


@jax.jit
def kernel(indices, weight, scales):
    B, L = indices.shape
    V, D = weight.shape
    NG = scales.shape[1]
    N = B * L
    idx_flat = indices.reshape(N)
    w32 = lax.bitcast_convert_type(
        weight.reshape(V, D // 4, 4), jnp.int32)
    wblk = w32.reshape(V * (D // 4) // 128, 128)
    sclf = scales.reshape(V * NG)
    fn = _build_kernel(N, V, D, NG, C=128)
    out = fn(idx_flat, wblk, sclf)
    return out.reshape(B, L, D)


# speedup vs baseline: 1.0611x; 1.0611x over previous
import functools

import jax
import jax.numpy as jnp
from jax import lax
from jax.experimental import pallas as pl
from jax.experimental.pallas import tpu as pltpu
from jax.experimental.pallas import tpu_sc as plsc

_NC = 2
_NS = 16
_LANES = 16


def _build_kernel(N, V, D, NG, C):
    NW = _NC * _NS
    rows_per_w = N // NW
    n_chunks = rows_per_w // C
    W = D // 4
    RPB = 128 // W

    mesh = plsc.VectorSubcoreMesh(
        core_axis_name="c", subcore_axis_name="s",
        num_cores=_NC, num_subcores=_NS)

    buf = lambda shape, dtype: [pltpu.VMEM(shape, dtype) for _ in range(2)]

    @functools.partial(
        pl.kernel,
        out_type=jax.ShapeDtypeStruct((N * D,), jnp.float32),
        mesh=mesh,
        compiler_params=pltpu.CompilerParams(needs_layout_passes=False),
        scratch_types=[
            pltpu.VMEM((rows_per_w + _LANES,), jnp.int32),
            buf((C,), jnp.int32),
            buf((NG * C,), jnp.int32),
            buf((C, 128), jnp.int32),
            buf((NG * C,), jnp.float32),
            buf((C * D,), jnp.float32),
            [pltpu.SemaphoreType.DMA for _ in range(2)],
            [pltpu.SemaphoreType.DMA for _ in range(2)],
            [pltpu.SemaphoreType.DMA for _ in range(2)],
        ],
    )
    def dequant(idx_hbm, wblk_hbm, sclf_hbm, out_hbm,
                idx_v, bidx_v, eidx_v, rows_v, scl_v, out_v,
                gsem, ssem, osem):
        wid = lax.axis_index("s") * _NC + lax.axis_index("c")
        base = wid * rows_per_w
        pltpu.sync_copy(idx_hbm.at[pl.ds(base, rows_per_w)],
                        idx_v.at[pl.ds(0, rows_per_w)])

        lane = lax.iota(jnp.int32, _LANES)
        col4 = lane * 4
        glo = lane >> 3
        r_in_reg = lane >> 2
        sub4 = lane & 3

        def start_gathers(c, b):
            cbase = c * C

            @plsc.parallel_loop(0, C // _LANES, unroll=4)
            def build_bidx(t):
                iv = idx_v[pl.ds(cbase + t * _LANES, _LANES)]
                bidx_v[b][pl.ds(t * _LANES, _LANES)] = iv >> 2

            @plsc.parallel_loop(0, NG * C // _LANES, unroll=4)
            def build_eidx(t):
                iv = idx_v[pl.ds(cbase + t * 4, _LANES)]
                gi = jnp.take(iv, r_in_reg, axis=0)
                eidx_v[b][pl.ds(t * _LANES, _LANES)] = gi * NG + sub4

            pltpu.async_copy(wblk_hbm.at[bidx_v[b]], rows_v[b], gsem[b])
            pltpu.async_copy(sclf_hbm.at[eidx_v[b]], scl_v[b], ssem[b])

        def compute_chunk(c, b):
            cbase = c * C

            @plsc.parallel_loop(0, C, unroll=4)
            def do_row(r):
                ivv = idx_v[pl.ds(cbase + r, _LANES)]
                sub32 = (ivv[0] & (RPB - 1)) * W
                rb = jnp.full((_LANES,), r * D, jnp.int32) + col4
                sb = jnp.full((_LANES,), r * NG, jnp.int32) + glo
                s_lo = plsc.load_gather(scl_v[b], [sb])
                s_hi = plsc.load_gather(scl_v[b], [sb + 2])
                w0 = rows_v[b][r, pl.ds(sub32, _LANES)]
                w1 = rows_v[b][r, pl.ds(sub32 + _LANES, _LANES)]
                for h, (w, s) in enumerate(((w0, s_lo), (w1, s_hi))):
                    for j in range(4):
                        v = (w << (24 - 8 * j)) >> 24
                        y = v.astype(jnp.float32) * s
                        plsc.store_scatter(out_v[b], [rb + (64 * h + j)], y)

        start_gathers(0, 0)

        def do_pair(g2, _):
            for b in range(2):
                c = g2 * 2 + b
                pltpu.make_async_copy(
                    wblk_hbm.at[bidx_v[b]], rows_v[b], gsem[b]).wait()
                pltpu.make_async_copy(
                    sclf_hbm.at[eidx_v[b]], scl_v[b], ssem[b]).wait()

                @pl.when(c + 1 < n_chunks)
                def _():
                    start_gathers(c + 1, 1 - b)

                @pl.when(c >= 2)
                def _():
                    pltpu.make_async_copy(
                        out_v[b],
                        out_hbm.at[pl.ds((base + (c - 2) * C) * D, C * D)],
                        osem[b]).wait()

                compute_chunk(c, b)
                pltpu.async_copy(
                    out_v[b], out_hbm.at[pl.ds((base + c * C) * D, C * D)],
                    osem[b])
            return 0

        lax.fori_loop(0, n_chunks // 2, do_pair, 0, unroll=False)

        for b in range(2):
            c = n_chunks - 2 + b
            pltpu.make_async_copy(
                out_v[b], out_hbm.at[pl.ds((base + c * C) * D, C * D)],
                osem[b]).wait()

    return dequant


@jax.jit
def kernel(indices, weight, scales):
    B, L = indices.shape
    V, D = weight.shape
    NG = scales.shape[1]
    N = B * L
    idx_flat = indices.reshape(N)
    w32 = lax.bitcast_convert_type(
        weight.reshape(V, D // 4, 4), jnp.int32)
    wblk = w32.reshape(V * (D // 4) // 128, 128)
    sclf = scales.reshape(V * NG)
    fn = _build_kernel(N, V, D, NG, C=128)
    out = fn(idx_flat, wblk, sclf)
    return out.reshape(B, L, D)


# speedup vs baseline: 1.9150x; 1.8048x over previous
import functools

import jax
import jax.numpy as jnp
from jax import lax
from jax.experimental import pallas as pl
from jax.experimental.pallas import tpu as pltpu
from jax.experimental.pallas import tpu_sc as plsc

_NC = 2
_NS = 16
_LANES = 16


def _build_kernel(N, V, D, NG, C):
    NW = _NC * _NS
    rows_per_w = N // NW
    n_chunks = rows_per_w // C
    W = D // 4
    RPB = 128 // W

    mesh = plsc.VectorSubcoreMesh(
        core_axis_name="c", subcore_axis_name="s",
        num_cores=_NC, num_subcores=_NS)

    buf = lambda shape, dtype: [pltpu.VMEM(shape, dtype) for _ in range(2)]

    @functools.partial(
        pl.kernel,
        out_type=jax.ShapeDtypeStruct((N * D,), jnp.float32),
        mesh=mesh,
        compiler_params=pltpu.CompilerParams(needs_layout_passes=False),
        scratch_types=[
            pltpu.VMEM((rows_per_w + _LANES,), jnp.int32),
            buf((C,), jnp.int32),
            buf((NG * C,), jnp.int32),
            buf((C, 128), jnp.int32),
            buf((NG * C + _LANES,), jnp.float32),
            buf((C * D,), jnp.float32),
            [pltpu.SemaphoreType.DMA for _ in range(2)],
            [pltpu.SemaphoreType.DMA for _ in range(2)],
            [pltpu.SemaphoreType.DMA for _ in range(2)],
        ],
    )
    def dequant(idx_hbm, w8_hbm, sclf_hbm, out_hbm,
                idx_v, bidx_v, eidx_v, rows_v, scl_v, out_v,
                gsem, ssem, osem):
        wblk_hbm = w8_hbm.bitcast(jnp.int32).reshape(V * D // 4 // 128, 128)
        wid = lax.axis_index("s") * _NC + lax.axis_index("c")
        base = wid * rows_per_w
        pltpu.sync_copy(idx_hbm.at[pl.ds(base, rows_per_w)],
                        idx_v.at[pl.ds(0, rows_per_w)])

        lane = lax.iota(jnp.int32, _LANES)
        col4 = lane * 4
        glo = lane >> 3
        r_in_reg = lane >> 2
        sub4 = lane & 3

        def start_gathers(c, b):
            cbase = c * C

            @plsc.parallel_loop(0, C // _LANES, unroll=4)
            def build_bidx(t):
                iv = idx_v[pl.ds(cbase + t * _LANES, _LANES)]
                bidx_v[b][pl.ds(t * _LANES, _LANES)] = iv >> 2

            @plsc.parallel_loop(0, NG * C // _LANES, unroll=4)
            def build_eidx(t):
                iv = idx_v[pl.ds(cbase + t * 4, _LANES)]
                gi = jnp.take(iv, r_in_reg, axis=0)
                eidx_v[b][pl.ds(t * _LANES, _LANES)] = gi * NG + sub4

            pltpu.async_copy(wblk_hbm.at[bidx_v[b]], rows_v[b], gsem[b])
            pltpu.async_copy(sclf_hbm.at[eidx_v[b]],
                             scl_v[b].at[pl.ds(0, NG * C)], ssem[b])

        def compute_chunk(c, b):
            cbase = c * C

            @plsc.parallel_loop(0, C, unroll=4)
            def do_row(r):
                ivv = idx_v[pl.ds(cbase + r, _LANES)]
                shl = 24 - 8 * (ivv[0] & (RPB - 1))
                sv = scl_v[b][pl.ds(r * NG, _LANES)]
                for t in range(8):
                    s = jnp.take(sv, jnp.full((_LANES,), t // 2, jnp.int32),
                                 axis=0)
                    w = rows_v[b][r, pl.ds(t * _LANES, _LANES)]
                    v = (w << shl) >> 24
                    y = v.astype(jnp.float32) * s
                    out_v[b][pl.ds(r * D + t * _LANES, _LANES)] = y

        start_gathers(0, 0)

        def do_pair(g2, _):
            for b in range(2):
                c = g2 * 2 + b
                pltpu.make_async_copy(
                    wblk_hbm.at[bidx_v[b]], rows_v[b], gsem[b]).wait()
                pltpu.make_async_copy(
                    sclf_hbm.at[eidx_v[b]],
                    scl_v[b].at[pl.ds(0, NG * C)], ssem[b]).wait()

                @pl.when(c + 1 < n_chunks)
                def _():
                    start_gathers(c + 1, 1 - b)

                @pl.when(c >= 2)
                def _():
                    pltpu.make_async_copy(
                        out_v[b],
                        out_hbm.at[pl.ds((base + (c - 2) * C) * D, C * D)],
                        osem[b]).wait()

                compute_chunk(c, b)
                pltpu.async_copy(
                    out_v[b], out_hbm.at[pl.ds((base + c * C) * D, C * D)],
                    osem[b])
            return 0

        lax.fori_loop(0, n_chunks // 2, do_pair, 0, unroll=False)

        for b in range(2):
            c = n_chunks - 2 + b
            pltpu.make_async_copy(
                out_v[b], out_hbm.at[pl.ds((base + c * C) * D, C * D)],
                osem[b]).wait()

    return dequant


@jax.jit
def kernel(indices, weight, scales):
    B, L = indices.shape
    V, D = weight.shape
    NG = scales.shape[1]
    N = B * L
    idx_flat = indices.reshape(N)
    sclf = scales.reshape(V * NG)
    fn = _build_kernel(N, V, D, NG, C=128)
    out = fn(idx_flat, weight, sclf)
    return out.reshape(B, L, D)
